# baseline (speedup 1.0000x reference)
import functools

import jax
import jax.numpy as jnp
from jax import lax
from jax.experimental import pallas as pl
from jax.experimental.pallas import tpu as pltpu
from jax.experimental.pallas import tpu_sc as plsc

N = 320000
D = 128
SEGS = 16
EPS = 1e-06

TCBLK = 1280
TCN = 307200
NTCB = TCN // TCBLK

NC = 2
NS = 16
NW = NC * NS
SC_ROWS = N - TCN
ROWS_PER_TILE = SC_ROWS // NW
R = 400
NCHUNK = ROWS_PER_TILE // R
GROUPS = R // 16
DCH = D // 16

assert TCN % TCBLK == 0 and N % TCBLK == 0
assert SC_ROWS % NW == 0 and ROWS_PER_TILE % R == 0

_mesh = plsc.VectorSubcoreMesh(core_axis_name="c", subcore_axis_name="s")


@functools.partial(
    pl.kernel,
    out_type=[
        jax.ShapeDtypeStruct((NW, SEGS, D), jnp.float32),
        jax.ShapeDtypeStruct((NW, SEGS), jnp.float32),
    ],
    mesh=_mesh,
    scratch_types=[
        pltpu.VMEM((R, D), jnp.float32),
        pltpu.VMEM((R, D), jnp.float32),
        pltpu.VMEM((R,), jnp.int32),
        pltpu.VMEM((R,), jnp.int32),
        pltpu.VMEM((SEGS, D), jnp.float32),
        pltpu.VMEM((SEGS,), jnp.float32),
        pltpu.SemaphoreType.DMA,
        pltpu.SemaphoreType.DMA,
    ],
)
def _seg_sums(x_hbm, ids_hbm, psum_hbm, pcnt_hbm,
              xbuf0, xbuf1, idbuf0, idbuf1, acc, cntv, sem0, sem1):
    cid = lax.axis_index("c")
    sid = lax.axis_index("s")
    wid = sid * NC + cid
    base = TCN + wid * ROWS_PER_TILE

    zero16 = jnp.zeros((16,), jnp.float32)
    lanes = lax.iota(jnp.int32, 16)

    def zero_acc(s, carry):
        for c in range(DCH):
            acc[s, pl.ds(c * 16, 16)] = zero16
        return carry

    lax.fori_loop(0, SEGS, zero_acc, 0)
    cntv[...] = zero16

    def make_group_processor(xbuf, idbuf):
        def accum_row(r, seg):
            for c in range(DCH):
                v = jnp.maximum(xbuf[r, pl.ds(c * 16, 16)], EPS)
                acc[seg, pl.ds(c * 16, 16)] += v

        def process_group(g, carry):
            r0 = g * 16
            ids = idbuf[pl.ds(r0, 16)]
            mn = ids[0]
            mx = ids[15]

            @pl.when(mn == mx)
            def _common():
                sums = [zero16 for _ in range(DCH)]
                for j in range(16):
                    for c in range(DCH):
                        v = jnp.maximum(xbuf[r0 + j, pl.ds(c * 16, 16)], EPS)
                        sums[c] = sums[c] + v
                for c in range(DCH):
                    acc[mn, pl.ds(c * 16, 16)] += sums[c]
                cntv[...] += jnp.where(lanes == mn, 16.0, 0.0)

            @pl.when(mn != mx)
            def _boundary():
                for j in range(16):
                    sj = ids[j]
                    accum_row(r0 + j, sj)
                    cntv[...] += jnp.where(lanes == sj, 1.0, 0.0)

            return carry

        return process_group

    process0 = make_group_processor(xbuf0, idbuf0)
    process1 = make_group_processor(xbuf1, idbuf1)

    def start(i, xbuf, idbuf, sem):
        row0 = base + i * R
        pltpu.async_copy(x_hbm.at[pl.ds(row0, R)], xbuf, sem)
        pltpu.async_copy(ids_hbm.at[pl.ds(row0, R)], idbuf, sem)

    def wait(i, xbuf, idbuf, sem):
        row0 = base + i * R
        pltpu.make_async_copy(x_hbm.at[pl.ds(row0, R)], xbuf, sem).wait()
        pltpu.make_async_copy(ids_hbm.at[pl.ds(row0, R)], idbuf, sem).wait()

    start(0, xbuf0, idbuf0, sem0)
    if NCHUNK > 1:
        start(1, xbuf1, idbuf1, sem1)

    def pair_body(p, carry):
        i0 = 2 * p
        wait(i0, xbuf0, idbuf0, sem0)
        lax.fori_loop(0, GROUPS, process0, 0)

        @pl.when(i0 + 2 < NCHUNK)
        def _():
            start(i0 + 2, xbuf0, idbuf0, sem0)

        wait(i0 + 1, xbuf1, idbuf1, sem1)
        lax.fori_loop(0, GROUPS, process1, 0)

        @pl.when(i0 + 3 < NCHUNK)
        def _():
            start(i0 + 3, xbuf1, idbuf1, sem1)

        return carry

    lax.fori_loop(0, NCHUNK // 2, pair_body, 0)
    if NCHUNK % 2:
        wait(NCHUNK - 1, xbuf0, idbuf0, sem0)
        lax.fori_loop(0, GROUPS, process0, 0)

    pltpu.sync_copy(acc, psum_hbm.at[wid])
    pltpu.sync_copy(cntv, pcnt_hbm.at[wid])


def _tc_body(ids_ref, x_ref, out_ref, cnt_ref):
    @pl.when(pl.program_id(0) == 0)
    def _init():
        out_ref[...] = jnp.zeros_like(out_ref)
        cnt_ref[...] = jnp.zeros_like(cnt_ref)

    x = jnp.maximum(x_ref[...], EPS)
    ids = ids_ref[0]
    mn = jnp.min(ids)
    mx = jnp.max(ids)
    seg_rows = lax.broadcasted_iota(jnp.int32, (SEGS, D), 0)

    @pl.when(mn == mx)
    def _uniform():
        out_ref[pl.ds(mn, 1), :] += jnp.sum(x, axis=0, keepdims=True)
        cnt_ref[...] += jnp.where(seg_rows == mn, float(TCBLK), 0.0)

    @pl.when(mn != mx)
    def _boundary():
        onehot = (lax.broadcasted_iota(jnp.int32, (SEGS, TCBLK), 0)
                  == ids).astype(jnp.float32)
        out_ref[...] += jnp.dot(onehot, x,
                                preferred_element_type=jnp.float32)
        cnt_ref[...] += jnp.broadcast_to(
            jnp.sum(onehot, axis=1)[:, None], (SEGS, D))


_tc_partial = pl.pallas_call(
    _tc_body,
    grid=(NTCB,),
    in_specs=[
        pl.BlockSpec((1, 1, TCBLK), lambda i: (i, 0, 0)),
        pl.BlockSpec((TCBLK, D), lambda i: (i, 0)),
    ],
    out_specs=[
        pl.BlockSpec((SEGS, D), lambda i: (0, 0)),
        pl.BlockSpec((SEGS, D), lambda i: (0, 0)),
    ],
    out_shape=[
        jax.ShapeDtypeStruct((SEGS, D), jnp.float32),
        jax.ShapeDtypeStruct((SEGS, D), jnp.float32),
    ],
)


def _combine_body(psum_ref, pcnt_ref, tcs_ref, tcc_ref, out_ref):
    sums = jnp.sum(psum_ref[...], axis=0) + tcs_ref[...]
    cnts = jnp.sum(pcnt_ref[...], axis=0)[:, None] + tcc_ref[:, 0:1]
    out_ref[...] = sums / jnp.maximum(cnts, 1.0)


_combine = pl.pallas_call(
    _combine_body,
    out_shape=jax.ShapeDtypeStruct((SEGS, D), jnp.float32),
)


def kernel(x_feat, segment_ids, num_segments):
    ids = segment_ids.astype(jnp.int32)
    psum, pcnt = _seg_sums(x_feat, ids)
    ids3 = ids.reshape(N // TCBLK, 1, TCBLK)
    tcs, tcc = _tc_partial(ids3, x_feat)
    return _combine(psum, pcnt, tcs, tcc)


# speedup vs baseline: 2.3039x; 2.3039x over previous
import functools

import jax
import jax.numpy as jnp
from jax import lax
from jax.experimental import pallas as pl
from jax.experimental.pallas import tpu as pltpu
from jax.experimental.pallas import tpu_sc as plsc

N = 320000
D = 128
SEGS = 16
EPS = 1e-06

TCBLK = 1280
TCN = 102400
NTCB = TCN // TCBLK

NC = 2
NS = 16
NW = NC * NS
SC_ROWS = N - TCN
ROWS_PER_TILE = SC_ROWS // NW
R = 400
NCHUNK = ROWS_PER_TILE // R
GROUPS = R // 16
DCH = D // 16

assert TCN % TCBLK == 0 and N % TCBLK == 0
assert SC_ROWS % NW == 0 and ROWS_PER_TILE % R == 0

_mesh = plsc.VectorSubcoreMesh(core_axis_name="c", subcore_axis_name="s")


@functools.partial(
    pl.kernel,
    out_type=[
        jax.ShapeDtypeStruct((NW, SEGS, D), jnp.float32),
        jax.ShapeDtypeStruct((NW, SEGS), jnp.float32),
    ],
    mesh=_mesh,
    scratch_types=[
        pltpu.VMEM((R, D), jnp.float32),
        pltpu.VMEM((R, D), jnp.float32),
        pltpu.VMEM((R,), jnp.int32),
        pltpu.VMEM((R,), jnp.int32),
        pltpu.VMEM((SEGS, D), jnp.float32),
        pltpu.VMEM((SEGS,), jnp.float32),
        pltpu.SemaphoreType.DMA,
        pltpu.SemaphoreType.DMA,
    ],
)
def _seg_sums(x_hbm, ids_hbm, psum_hbm, pcnt_hbm,
              xbuf0, xbuf1, idbuf0, idbuf1, acc, cntv, sem0, sem1):
    cid = lax.axis_index("c")
    sid = lax.axis_index("s")
    wid = sid * NC + cid
    base = TCN + wid * ROWS_PER_TILE

    zero16 = jnp.zeros((16,), jnp.float32)
    lanes = lax.iota(jnp.int32, 16)

    def zero_acc(s, carry):
        for c in range(DCH):
            acc[s, pl.ds(c * 16, 16)] = zero16
        return carry

    lax.fori_loop(0, SEGS, zero_acc, 0)
    cntv[...] = zero16

    def make_group_processor(xbuf, idbuf):
        def accum_row(r, seg):
            for c in range(DCH):
                v = jnp.maximum(xbuf[r, pl.ds(c * 16, 16)], EPS)
                acc[seg, pl.ds(c * 16, 16)] += v

        def process_group(g, carry):
            r0 = g * 16
            ids = idbuf[pl.ds(r0, 16)]
            mn = ids[0]
            mx = ids[15]

            @pl.when(mn == mx)
            def _common():
                sums = [zero16 for _ in range(DCH)]
                for j in range(16):
                    for c in range(DCH):
                        v = jnp.maximum(xbuf[r0 + j, pl.ds(c * 16, 16)], EPS)
                        sums[c] = sums[c] + v
                for c in range(DCH):
                    acc[mn, pl.ds(c * 16, 16)] += sums[c]
                cntv[...] += jnp.where(lanes == mn, 16.0, 0.0)

            @pl.when(mn != mx)
            def _boundary():
                for j in range(16):
                    sj = ids[j]
                    accum_row(r0 + j, sj)
                    cntv[...] += jnp.where(lanes == sj, 1.0, 0.0)

            return carry

        return process_group

    process0 = make_group_processor(xbuf0, idbuf0)
    process1 = make_group_processor(xbuf1, idbuf1)

    def start(i, xbuf, idbuf, sem):
        row0 = base + i * R
        pltpu.async_copy(x_hbm.at[pl.ds(row0, R)], xbuf, sem)
        pltpu.async_copy(ids_hbm.at[pl.ds(row0, R)], idbuf, sem)

    def wait(i, xbuf, idbuf, sem):
        row0 = base + i * R
        pltpu.make_async_copy(x_hbm.at[pl.ds(row0, R)], xbuf, sem).wait()
        pltpu.make_async_copy(ids_hbm.at[pl.ds(row0, R)], idbuf, sem).wait()

    start(0, xbuf0, idbuf0, sem0)
    if NCHUNK > 1:
        start(1, xbuf1, idbuf1, sem1)

    def pair_body(p, carry):
        i0 = 2 * p
        wait(i0, xbuf0, idbuf0, sem0)
        lax.fori_loop(0, GROUPS, process0, 0)

        @pl.when(i0 + 2 < NCHUNK)
        def _():
            start(i0 + 2, xbuf0, idbuf0, sem0)

        wait(i0 + 1, xbuf1, idbuf1, sem1)
        lax.fori_loop(0, GROUPS, process1, 0)

        @pl.when(i0 + 3 < NCHUNK)
        def _():
            start(i0 + 3, xbuf1, idbuf1, sem1)

        return carry

    lax.fori_loop(0, NCHUNK // 2, pair_body, 0)
    if NCHUNK % 2:
        wait(NCHUNK - 1, xbuf0, idbuf0, sem0)
        lax.fori_loop(0, GROUPS, process0, 0)

    pltpu.sync_copy(acc, psum_hbm.at[wid])
    pltpu.sync_copy(cntv, pcnt_hbm.at[wid])


def _tc_body(ids_ref, x_ref, out_ref, cnt_ref):
    @pl.when(pl.program_id(0) == 0)
    def _init():
        out_ref[...] = jnp.zeros_like(out_ref)
        cnt_ref[...] = jnp.zeros_like(cnt_ref)

    x = jnp.maximum(x_ref[...], EPS)
    onehot = (lax.broadcasted_iota(jnp.int32, (SEGS, TCBLK), 0)
              == ids_ref[0]).astype(jnp.float32)
    out_ref[...] += jnp.dot(onehot, x, preferred_element_type=jnp.float32)
    cnt_ref[...] += jnp.broadcast_to(
        jnp.sum(onehot, axis=1)[:, None], (SEGS, D))


_tc_partial = pl.pallas_call(
    _tc_body,
    grid=(NTCB,),
    in_specs=[
        pl.BlockSpec((1, 1, TCBLK), lambda i: (i, 0, 0)),
        pl.BlockSpec((TCBLK, D), lambda i: (i, 0)),
    ],
    out_specs=[
        pl.BlockSpec((SEGS, D), lambda i: (0, 0)),
        pl.BlockSpec((SEGS, D), lambda i: (0, 0)),
    ],
    out_shape=[
        jax.ShapeDtypeStruct((SEGS, D), jnp.float32),
        jax.ShapeDtypeStruct((SEGS, D), jnp.float32),
    ],
)


def _combine_body(psum_ref, pcnt_ref, tcs_ref, tcc_ref, out_ref):
    sums = jnp.sum(psum_ref[...], axis=0) + tcs_ref[...]
    cnts = jnp.sum(pcnt_ref[...], axis=0)[:, None] + tcc_ref[:, 0:1]
    out_ref[...] = sums / jnp.maximum(cnts, 1.0)


_combine = pl.pallas_call(
    _combine_body,
    out_shape=jax.ShapeDtypeStruct((SEGS, D), jnp.float32),
)


def kernel(x_feat, segment_ids, num_segments):
    ids = segment_ids.astype(jnp.int32)
    psum, pcnt = _seg_sums(x_feat, ids)
    ids3 = ids.reshape(N // TCBLK, 1, TCBLK)
    tcs, tcc = _tc_partial(ids3, x_feat)
    return _combine(psum, pcnt, tcs, tcc)
